# baseline (speedup 1.0000x reference)
import functools

import jax
import jax.numpy as jnp
from jax import lax
from jax.experimental import pallas as pl
from jax.experimental.pallas import tpu as pltpu
from jax.experimental.pallas import tpu_sc as plsc

_NUM_CORES = 2
_NUM_SUBCORES = 16
_NUM_WORKERS = _NUM_CORES * _NUM_SUBCORES
_RPW = 128
_LANES = 16


def _gather_transposed(table, idx):
    batch, hist = idx.shape
    embed_dim = table.shape[1]
    assert batch == _RPW * _NUM_WORKERS and embed_dim % 8 == 0 and hist % 2 == 0

    mesh = plsc.VectorSubcoreMesh(core_axis_name="c", subcore_axis_name="s")

    @functools.partial(
        pl.kernel,
        mesh=mesh,
        out_type=jax.ShapeDtypeStruct(
            (hist, embed_dim // 8, _NUM_WORKERS, 8 * _RPW), table.dtype),
        scratch_types=[
            pltpu.VMEM((_RPW, hist), jnp.int32),
            pltpu.VMEM((_RPW,), jnp.int32),
            pltpu.VMEM((_RPW,), jnp.int32),
            pltpu.VMEM((_RPW, embed_dim), table.dtype),
            pltpu.VMEM((_RPW, embed_dim), table.dtype),
            pltpu.VMEM((embed_dim // 8, 8 * _RPW), table.dtype),
            pltpu.VMEM((embed_dim // 8, 8 * _RPW), table.dtype),
            pltpu.SemaphoreType.DMA,
            pltpu.SemaphoreType.DMA,
            pltpu.SemaphoreType.DMA,
            pltpu.SemaphoreType.DMA,
        ],
        compiler_params=pltpu.CompilerParams(use_tc_tiling_on_sc=False,
                                             needs_layout_passes=False),
    )
    def k(table_hbm, idx_hbm, out_hbm, idx_blk, col0, col1, g0, g1, gt0, gt1,
          sg0, sg1, so0, so1):
        wid = lax.axis_index("s") * _NUM_CORES + lax.axis_index("c")
        cols = (col0, col1)
        grows = (g0, g1)
        gtrans = (gt0, gt1)
        sg = (sg0, sg1)
        so = (so0, so1)

        iota = lax.iota(jnp.int32, _LANES)
        izero = iota * 0
        rowv = [_LANES * j + iota for j in range(_RPW // _LANES)]

        def extract(h, p):
            hv = izero + h
            for j in range(_RPW // _LANES):
                cols[p][pl.ds(_LANES * j, _LANES)] = plsc.load_gather(
                    idx_blk, [rowv[j], hv])

        def start_gather(p):
            return pltpu.async_copy(table_hbm.at[cols[p]], grows[p], sg[p])

        pltpu.sync_copy(idx_hbm.at[pl.ds(wid * _RPW, _RPW)], idx_blk)

        extract(0, 0)
        start_gather(0)
        extract(1, 1)
        start_gather(1)

        @pl.loop(0, hist, step=2)
        def _(h0):
            for p in (0, 1):
                h = h0 + p
                pltpu.make_async_copy(table_hbm.at[cols[p]], grows[p],
                                      sg[p]).wait()

                @pl.when(h >= 2)
                def _():
                    pltpu.make_async_copy(gtrans[p], out_hbm.at[0, :, wid, :],
                                          so[p]).wait()

                for c in range(embed_dim):
                    cv = izero + c
                    ct, r = c // 8, c % 8
                    for j in range(_RPW // _LANES):
                        gtrans[p][ct, pl.ds(r * _RPW + _LANES * j, _LANES)] = (
                            plsc.load_gather(grows[p], [rowv[j], cv]))

                pltpu.async_copy(gtrans[p], out_hbm.at[h, :, wid, :], so[p])

                @pl.when(h + 2 < hist)
                def _():
                    extract(h + 2, p)
                    start_gather(p)

        pltpu.make_async_copy(gt0, out_hbm.at[0, :, wid, :], so0).wait()
        pltpu.make_async_copy(gt1, out_hbm.at[0, :, wid, :], so1).wait()

    return k(table, idx)


@jax.jit
def kernel(input, table):
    batch, hist = input.shape
    embed_dim = table.shape[1]
    out5d = _gather_transposed(table, input.astype(jnp.int32)).reshape(
        hist, embed_dim // 8, _NUM_WORKERS, 8, _RPW)
    return out5d.transpose(2, 4, 0, 1, 3).reshape(batch, hist, embed_dim)


# speedup vs baseline: 1.2397x; 1.2397x over previous
import functools

import jax
import jax.numpy as jnp
from jax import lax
from jax.experimental import pallas as pl
from jax.experimental.pallas import tpu as pltpu
from jax.experimental.pallas import tpu_sc as plsc

_NUM_CORES = 2
_NUM_SUBCORES = 16
_NUM_WORKERS = _NUM_CORES * _NUM_SUBCORES
_RPW = 128
_LANES = 16


def _gather_transposed(table, idx):
    batch, hist = idx.shape
    embed_dim = table.shape[1]
    assert batch == _RPW * _NUM_WORKERS and embed_dim % 8 == 0 and hist % 2 == 0

    mesh = plsc.VectorSubcoreMesh(core_axis_name="c", subcore_axis_name="s")

    @functools.partial(
        pl.kernel,
        mesh=mesh,
        out_type=jax.ShapeDtypeStruct(
            (hist, embed_dim // 8, _NUM_WORKERS, 8 * _RPW), table.dtype),
        scratch_types=[
            pltpu.VMEM((_RPW, hist), jnp.int32),
            pltpu.VMEM((_RPW,), jnp.int32),
            pltpu.VMEM((_RPW,), jnp.int32),
            pltpu.VMEM((_RPW, embed_dim), table.dtype),
            pltpu.VMEM((_RPW, embed_dim), table.dtype),
            pltpu.VMEM((embed_dim // 8, 8 * _RPW), table.dtype),
            pltpu.VMEM((embed_dim // 8, 8 * _RPW), table.dtype),
            pltpu.SemaphoreType.DMA,
            pltpu.SemaphoreType.DMA,
            pltpu.SemaphoreType.DMA,
            pltpu.SemaphoreType.DMA,
        ],
        compiler_params=pltpu.CompilerParams(use_tc_tiling_on_sc=False,
                                             needs_layout_passes=False),
    )
    def k(table_hbm, idx_hbm, out_hbm, idx_blk, col0, col1, g0, g1, gt0, gt1,
          sg0, sg1, so0, so1):
        wid = lax.axis_index("s") * _NUM_CORES + lax.axis_index("c")
        cols = (col0, col1)
        grows = (g0, g1)
        gtrans = (gt0, gt1)
        sg = (sg0, sg1)
        so = (so0, so1)

        iota = lax.iota(jnp.int32, _LANES)
        izero = iota * 0
        rowv = [_LANES * j + iota for j in range(_RPW // _LANES)]

        def extract(h, p):
            hv = izero + h
            for j in range(_RPW // _LANES):
                cols[p][pl.ds(_LANES * j, _LANES)] = plsc.load_gather(
                    idx_blk, [rowv[j], hv])

        def start_gather(p):
            return pltpu.async_copy(table_hbm.at[cols[p]], grows[p], sg[p])

        pltpu.sync_copy(idx_hbm.at[pl.ds(wid * _RPW, _RPW)], idx_blk)

        extract(0, 0)
        start_gather(0)
        extract(1, 1)
        start_gather(1)

        @pl.loop(0, hist, step=2)
        def _(h0):
            for p in (0, 1):
                h = h0 + p
                pltpu.make_async_copy(table_hbm.at[cols[p]], grows[p],
                                      sg[p]).wait()

                @pl.when(h >= 2)
                def _():
                    pltpu.make_async_copy(gtrans[p], out_hbm.at[0, :, wid, :],
                                          so[p]).wait()

                for c0 in range(0, embed_dim, 2):
                    vals = []
                    for c in (c0, c0 + 1):
                        cv = izero + c
                        for j in range(_RPW // _LANES):
                            vals.append((c, j, plsc.load_gather(
                                grows[p], [rowv[j], cv])))
                    for c, j, v in vals:
                        ct, r = c // 8, c % 8
                        gtrans[p][ct, pl.ds(r * _RPW + _LANES * j,
                                            _LANES)] = v

                pltpu.async_copy(gtrans[p], out_hbm.at[h, :, wid, :], so[p])

                @pl.when(h + 2 < hist)
                def _():
                    extract(h + 2, p)
                    start_gather(p)

        pltpu.make_async_copy(gt0, out_hbm.at[0, :, wid, :], so0).wait()
        pltpu.make_async_copy(gt1, out_hbm.at[0, :, wid, :], so1).wait()

    return k(table, idx)


@jax.jit
def kernel(input, table):
    batch, hist = input.shape
    embed_dim = table.shape[1]
    out5d = _gather_transposed(table, input.astype(jnp.int32)).reshape(
        hist, embed_dim // 8, _NUM_WORKERS, 8, _RPW)
    return out5d.transpose(2, 4, 0, 1, 3).reshape(batch, hist, embed_dim)


# speedup vs baseline: 1.3038x; 1.0517x over previous
import functools

import jax
import jax.numpy as jnp
from jax import lax
from jax.experimental import pallas as pl
from jax.experimental.pallas import tpu as pltpu
from jax.experimental.pallas import tpu_sc as plsc

_NUM_CORES = 2
_NUM_SUBCORES = 16
_NUM_WORKERS = _NUM_CORES * _NUM_SUBCORES
_RPW = 128
_LANES = 16


def _gather_transposed(table, idx):
    batch, hist = idx.shape
    embed_dim = table.shape[1]
    assert batch == _RPW * _NUM_WORKERS and embed_dim % 8 == 0 and hist % 2 == 0

    mesh = plsc.VectorSubcoreMesh(core_axis_name="c", subcore_axis_name="s")

    @functools.partial(
        pl.kernel,
        mesh=mesh,
        out_type=jax.ShapeDtypeStruct(
            (hist, embed_dim // 8, _NUM_WORKERS, 8 * _RPW), table.dtype),
        scratch_types=[
            pltpu.VMEM((_RPW, hist), jnp.int32),
            pltpu.VMEM((_RPW,), jnp.int32),
            pltpu.VMEM((_RPW,), jnp.int32),
            pltpu.VMEM((_RPW, embed_dim), table.dtype),
            pltpu.VMEM((_RPW, embed_dim), table.dtype),
            pltpu.VMEM((embed_dim // 8, 8 * _RPW), table.dtype),
            pltpu.VMEM((embed_dim // 8, 8 * _RPW), table.dtype),
            pltpu.SemaphoreType.DMA,
            pltpu.SemaphoreType.DMA,
            pltpu.SemaphoreType.DMA,
            pltpu.SemaphoreType.DMA,
        ],
        compiler_params=pltpu.CompilerParams(use_tc_tiling_on_sc=False,
                                             needs_layout_passes=False),
    )
    def k(table_hbm, idx_hbm, out_hbm, idx_blk, col0, col1, g0, g1, gt0, gt1,
          sg0, sg1, so0, so1):
        wid = lax.axis_index("s") * _NUM_CORES + lax.axis_index("c")
        cols = (col0, col1)
        grows = (g0, g1)
        gtrans = (gt0, gt1)
        sg = (sg0, sg1)
        so = (so0, so1)

        iota = lax.iota(jnp.int32, _LANES)
        izero = iota * 0
        rowv = [_LANES * j + iota for j in range(_RPW // _LANES)]

        def extract(h, p):
            hv = izero + h
            for j in range(_RPW // _LANES):
                cols[p][pl.ds(_LANES * j, _LANES)] = plsc.load_gather(
                    idx_blk, [rowv[j], hv])

        def start_gather(p):
            return pltpu.async_copy(table_hbm.at[cols[p]], grows[p], sg[p])

        pltpu.sync_copy(idx_hbm.at[pl.ds(wid * _RPW, _RPW)], idx_blk)

        extract(0, 0)
        start_gather(0)
        extract(1, 1)
        start_gather(1)

        @pl.loop(0, hist, step=2)
        def _(h0):
            for p in (0, 1):
                h = h0 + p
                pltpu.make_async_copy(table_hbm.at[cols[p]], grows[p],
                                      sg[p]).wait()

                @pl.when(h >= 2)
                def _():
                    pltpu.make_async_copy(gtrans[p], out_hbm.at[0, :, wid, :],
                                          so[p]).wait()

                @pl.loop(0, embed_dim // 2)
                def _(q):
                    vals = []
                    for dc in (0, 1):
                        c = 2 * q + dc
                        cv = izero + c
                        for j in range(_RPW // _LANES):
                            vals.append((c, j, plsc.load_gather(
                                grows[p], [rowv[j], cv])))
                    for c, j, v in vals:
                        ct = lax.div(c, 8)
                        r = lax.rem(c, 8)
                        gtrans[p][ct, pl.ds(r * _RPW + _LANES * j,
                                            _LANES)] = v

                pltpu.async_copy(gtrans[p], out_hbm.at[h, :, wid, :], so[p])

                @pl.when(h + 2 < hist)
                def _():
                    extract(h + 2, p)
                    start_gather(p)

        pltpu.make_async_copy(gt0, out_hbm.at[0, :, wid, :], so0).wait()
        pltpu.make_async_copy(gt1, out_hbm.at[0, :, wid, :], so1).wait()

    return k(table, idx)


@jax.jit
def kernel(input, table):
    batch, hist = input.shape
    embed_dim = table.shape[1]
    out5d = _gather_transposed(table, input.astype(jnp.int32)).reshape(
        hist, embed_dim // 8, _NUM_WORKERS, 8, _RPW)
    return out5d.transpose(2, 4, 0, 1, 3).reshape(batch, hist, embed_dim)


# speedup vs baseline: 4.0206x; 3.0838x over previous
import functools

import jax
import jax.numpy as jnp
from jax import lax
from jax.experimental import pallas as pl
from jax.experimental.pallas import tpu as pltpu
from jax.experimental.pallas import tpu_sc as plsc

_NUM_CORES = 2
_NUM_SUBCORES = 16
_NUM_WORKERS = _NUM_CORES * _NUM_SUBCORES
_RPW = 128
_LANES = 16


def _gather_transposed(table, idx):
    batch, hist = idx.shape
    embed_dim = table.shape[1]
    assert batch == _RPW * _NUM_WORKERS and embed_dim % 8 == 0 and hist % 2 == 0

    mesh = plsc.VectorSubcoreMesh(core_axis_name="c", subcore_axis_name="s")

    @functools.partial(
        pl.kernel,
        mesh=mesh,
        out_type=jax.ShapeDtypeStruct(
            (hist, embed_dim // 8, _NUM_WORKERS, 8 * _RPW), table.dtype),
        scratch_types=[
            pltpu.VMEM((_RPW, hist), jnp.int32),
            pltpu.VMEM((_RPW,), jnp.int32),
            pltpu.VMEM((_RPW,), jnp.int32),
            pltpu.VMEM((_RPW, embed_dim), table.dtype),
            pltpu.VMEM((_RPW, embed_dim), table.dtype),
            pltpu.VMEM((embed_dim // 8, 8 * _RPW), table.dtype),
            pltpu.VMEM((embed_dim // 8, 8 * _RPW), table.dtype),
            pltpu.VMEM((_RPW, embed_dim), table.dtype),
            pltpu.SemaphoreType.DMA,
            pltpu.SemaphoreType.DMA,
            pltpu.SemaphoreType.DMA,
            pltpu.SemaphoreType.DMA,
        ],
        compiler_params=pltpu.CompilerParams(use_tc_tiling_on_sc=False,
                                             needs_layout_passes=False),
    )
    def k(table_hbm, idx_hbm, out_hbm, idx_blk, col0, col1, g0, g1, gt0, gt1,
          skew, sg0, sg1, so0, so1):
        wid = lax.axis_index("s") * _NUM_CORES + lax.axis_index("c")
        cols = (col0, col1)
        grows = (g0, g1)
        gtrans = (gt0, gt1)
        sg = (sg0, sg1)
        so = (so0, so1)

        iota = lax.iota(jnp.int32, _LANES)
        izero = iota * 0
        rowv = [_LANES * j + iota for j in range(_RPW // _LANES)]

        def extract(h, p):
            hv = izero + h
            for j in range(_RPW // _LANES):
                cols[p][pl.ds(_LANES * j, _LANES)] = plsc.load_gather(
                    idx_blk, [rowv[j], hv])

        def start_gather(p):
            return pltpu.async_copy(table_hbm.at[cols[p]], grows[p], sg[p])

        pltpu.sync_copy(idx_hbm.at[pl.ds(wid * _RPW, _RPW)], idx_blk)

        extract(0, 0)
        start_gather(0)
        extract(1, 1)
        start_gather(1)

        @pl.loop(0, hist, step=2)
        def _(h0):
            for p in (0, 1):
                h = h0 + p
                pltpu.make_async_copy(table_hbm.at[cols[p]], grows[p],
                                      sg[p]).wait()

                @pl.when(h >= 2)
                def _():
                    pltpu.make_async_copy(gtrans[p], out_hbm.at[0, :, wid, :],
                                          so[p]).wait()

                @pl.loop(0, _RPW, step=4)
                def _(b0):
                    items = []
                    for db in range(4):
                        b = b0 + db
                        for k in range(embed_dim // _LANES):
                            v = grows[p][b, pl.ds(_LANES * k, _LANES)]
                            kv = (iota + (b + _LANES * k)) & (embed_dim - 1)
                            items.append((b, kv, v))
                    for b, kv, v in items:
                        plsc.store_scatter(skew, [izero + b, kv], v)

                @pl.loop(0, embed_dim // 2)
                def _(q):
                    vals = []
                    for dc in (0, 1):
                        c = 2 * q + dc
                        for j in range(_RPW // _LANES):
                            kv = (rowv[j] + c) & (embed_dim - 1)
                            vals.append((c, j, plsc.load_gather(
                                skew, [rowv[j], kv])))
                    for c, j, v in vals:
                        ct = lax.div(c, 8)
                        r = lax.rem(c, 8)
                        gtrans[p][ct, pl.ds(r * _RPW + _LANES * j,
                                            _LANES)] = v

                pltpu.async_copy(gtrans[p], out_hbm.at[h, :, wid, :], so[p])

                @pl.when(h + 2 < hist)
                def _():
                    extract(h + 2, p)
                    start_gather(p)

        pltpu.make_async_copy(gt0, out_hbm.at[0, :, wid, :], so0).wait()
        pltpu.make_async_copy(gt1, out_hbm.at[0, :, wid, :], so1).wait()

    return k(table, idx)


@jax.jit
def kernel(input, table):
    batch, hist = input.shape
    embed_dim = table.shape[1]
    out5d = _gather_transposed(table, input.astype(jnp.int32)).reshape(
        hist, embed_dim // 8, _NUM_WORKERS, 8, _RPW)
    return out5d.transpose(2, 4, 0, 1, 3).reshape(batch, hist, embed_dim)


# speedup vs baseline: 4.0467x; 1.0065x over previous
import functools

import jax
import jax.numpy as jnp
from jax import lax
from jax.experimental import pallas as pl
from jax.experimental.pallas import tpu as pltpu
from jax.experimental.pallas import tpu_sc as plsc

_NUM_CORES = 2
_NUM_SUBCORES = 16
_NUM_WORKERS = _NUM_CORES * _NUM_SUBCORES
_RPW = 128
_LANES = 16
_NBUF = 4


def _gather_transposed(table, idx):
    batch, hist = idx.shape
    embed_dim = table.shape[1]
    assert batch == _RPW * _NUM_WORKERS and embed_dim % 8 == 0
    assert hist % _NBUF == 0

    mesh = plsc.VectorSubcoreMesh(core_axis_name="c", subcore_axis_name="s")

    @functools.partial(
        pl.kernel,
        mesh=mesh,
        out_type=jax.ShapeDtypeStruct(
            (hist, embed_dim // 8, _NUM_WORKERS, 8 * _RPW), table.dtype),
        scratch_types=[
            pltpu.VMEM((_RPW, hist), jnp.int32),
        ] + [pltpu.VMEM((_RPW,), jnp.int32) for _ in range(_NBUF)] + [
            pltpu.VMEM((_RPW, embed_dim), table.dtype) for _ in range(_NBUF)
        ] + [
            pltpu.VMEM((embed_dim // 8, 8 * _RPW), table.dtype),
            pltpu.VMEM((embed_dim // 8, 8 * _RPW), table.dtype),
            pltpu.VMEM((_RPW, embed_dim), table.dtype),
        ] + [pltpu.SemaphoreType.DMA for _ in range(_NBUF + 2)],
        compiler_params=pltpu.CompilerParams(use_tc_tiling_on_sc=False,
                                             needs_layout_passes=False),
    )
    def k(table_hbm, idx_hbm, out_hbm, idx_blk,
          col0, col1, col2, col3, g0, g1, g2, g3, gt0, gt1, skew,
          sg0, sg1, sg2, sg3, so0, so1):
        wid = lax.axis_index("s") * _NUM_CORES + lax.axis_index("c")
        cols = (col0, col1, col2, col3)
        grows = (g0, g1, g2, g3)
        gtrans = (gt0, gt1)
        sg = (sg0, sg1, sg2, sg3)
        so = (so0, so1)

        iota = lax.iota(jnp.int32, _LANES)
        izero = iota * 0
        rowv = [_LANES * j + iota for j in range(_RPW // _LANES)]
        cmask = embed_dim - 1

        def extract(h, p):
            hv = izero + h
            for j in range(_RPW // _LANES):
                cols[p][pl.ds(_LANES * j, _LANES)] = plsc.load_gather(
                    idx_blk, [rowv[j], hv])

        def start_gather(p):
            return pltpu.async_copy(table_hbm.at[cols[p]], grows[p], sg[p])

        pltpu.sync_copy(idx_hbm.at[pl.ds(wid * _RPW, _RPW)], idx_blk)

        for p in range(_NBUF):
            extract(p, p)
            start_gather(p)

        @pl.loop(0, hist, step=_NBUF)
        def _(h0):
            for p in range(_NBUF):
                h = h0 + p
                q2 = p % 2
                pltpu.make_async_copy(table_hbm.at[cols[p]], grows[p],
                                      sg[p]).wait()

                @pl.when(h >= 2)
                def _():
                    pltpu.make_async_copy(gtrans[q2],
                                          out_hbm.at[0, :, wid, :],
                                          so[q2]).wait()

                @pl.loop(0, _RPW, step=4)
                def _(b0):
                    items = []
                    for db in range(4):
                        b = b0 + db
                        for kk in range(embed_dim // _LANES):
                            v = grows[p][b, pl.ds(_LANES * kk, _LANES)]
                            kv = (iota + (b + _LANES * kk)) & cmask
                            items.append((b, kv, v))
                    for b, kv, v in items:
                        plsc.store_scatter(skew, [izero + b, kv], v)

                @pl.loop(0, embed_dim // 2)
                def _(q):
                    vals = []
                    for dc in (0, 1):
                        c = 2 * q + dc
                        for j in range(_RPW // _LANES):
                            kv = (rowv[j] + c) & cmask
                            vals.append((c, j, plsc.load_gather(
                                skew, [rowv[j], kv])))
                    for c, j, v in vals:
                        ct = lax.div(c, 8)
                        r = lax.rem(c, 8)
                        gtrans[q2][ct, pl.ds(r * _RPW + _LANES * j,
                                             _LANES)] = v

                pltpu.async_copy(gtrans[q2], out_hbm.at[h, :, wid, :],
                                 so[q2])

                @pl.when(h + _NBUF < hist)
                def _():
                    extract(h + _NBUF, p)
                    start_gather(p)

        pltpu.make_async_copy(gt0, out_hbm.at[0, :, wid, :], so0).wait()
        pltpu.make_async_copy(gt1, out_hbm.at[0, :, wid, :], so1).wait()

    return k(table, idx)


@jax.jit
def kernel(input, table):
    batch, hist = input.shape
    embed_dim = table.shape[1]
    out5d = _gather_transposed(table, input.astype(jnp.int32)).reshape(
        hist, embed_dim // 8, _NUM_WORKERS, 8, _RPW)
    return out5d.transpose(2, 4, 0, 1, 3).reshape(batch, hist, embed_dim)
